# Initial kernel scaffold; baseline (speedup 1.0000x reference)
import jax
import jax.numpy as jnp
from jax.experimental import pallas as pl


def kernel(plan, W_pre, b_pre, codebook):
    raise NotImplementedError("write your pallas kernel here")


# speedup vs baseline: 1.1629x; 1.1629x over previous
import functools

import jax
import jax.numpy as jnp
from jax import lax
from jax.experimental import pallas as pl
from jax.experimental.pallas import tpu as pltpu
from jax.experimental.pallas import tpu_sc as plsc

TOK_BLK = 512
CB_BLK = 2048


def _dist_kernel(plan_ref, w_ref, b_ref, cbt_ref, idx_ref, msum_ref,
                 z_scr, rminA, rargA, rminB, rargB):
    i = pl.program_id(0)
    j = pl.program_id(1)

    @pl.when((i == 0) & (j == 0))
    def _():
        msum_ref[0, 0] = 0.0

    @pl.when(j == 0)
    def _():
        z = jnp.dot(plan_ref[...], w_ref[...],
                    preferred_element_type=jnp.float32) + b_ref[...]
        z_scr[...] = z

    z = z_scr[...]
    cbt = cbt_ref[...]
    zsq = jnp.sum(z * z, axis=1, keepdims=True)
    csq = jnp.sum(cbt * cbt, axis=0, keepdims=True)
    scores = jnp.dot(z, cbt, preferred_element_type=jnp.float32)
    dists = (zsq + csq) - 2.0 * scores
    lmin = jnp.min(dists, axis=1, keepdims=True)
    col = lax.broadcasted_iota(jnp.int32, (TOK_BLK, CB_BLK), 1)
    larg = jnp.min(jnp.where(dists == lmin, col, jnp.int32(2**30)),
                   axis=1, keepdims=True) + j * CB_BLK

    @pl.when(j == 0)
    def _():
        rminA[...] = lmin
        rargA[...] = larg

    @pl.when(j == 1)
    def _():
        better = lmin < rminA[...]
        rminA[...] = jnp.where(better, lmin, rminA[...])
        rargA[...] = jnp.where(better, larg, rargA[...])

    @pl.when(j == 2)
    def _():
        rminB[...] = lmin
        rargB[...] = larg

    @pl.when(j == 3)
    def _():
        better = lmin < rminB[...]
        mB = jnp.where(better, lmin, rminB[...])
        aB = jnp.where(better, larg, rargB[...])
        bits = lax.bitcast_convert_type(rminA[...], jnp.int32)
        rbits = (bits + 0x7FFF + ((bits >> 16) & 1)) & ~0xFFFF
        qA = lax.bitcast_convert_type(rbits, jnp.float32)
        upd = mB < qA
        idx_ref[...] = jnp.where(upd, aB, rargA[...])
        chosen = jnp.where(upd, mB, rminA[...])
        msum_ref[0, 0] += jnp.sum(chosen)


def _distance_argmin(plan2, w, b2, cbt, interpret=False):
    n, k = plan2.shape
    d, m = cbt.shape
    grid = (n // TOK_BLK, m // CB_BLK)
    return pl.pallas_call(
        _dist_kernel,
        grid=grid,
        in_specs=[
            pl.BlockSpec((TOK_BLK, k), lambda i, j: (i, 0)),
            pl.BlockSpec((k, d), lambda i, j: (0, 0)),
            pl.BlockSpec((1, d), lambda i, j: (0, 0)),
            pl.BlockSpec((d, CB_BLK), lambda i, j: (0, j)),
        ],
        out_specs=[
            pl.BlockSpec((TOK_BLK, 1), lambda i, j: (i, 0)),
            pl.BlockSpec((1, 1), lambda i, j: (0, 0),
                         memory_space=pltpu.SMEM),
        ],
        out_shape=[
            jax.ShapeDtypeStruct((n, 1), jnp.int32),
            jax.ShapeDtypeStruct((1, 1), jnp.float32),
        ],
        scratch_shapes=[
            pltpu.VMEM((TOK_BLK, d), jnp.float32),
            pltpu.VMEM((TOK_BLK, 1), jnp.float32),
            pltpu.VMEM((TOK_BLK, 1), jnp.int32),
            pltpu.VMEM((TOK_BLK, 1), jnp.float32),
            pltpu.VMEM((TOK_BLK, 1), jnp.int32),
        ],
        interpret=interpret,
    )(plan2, w, b2, cbt)


def _sc_gather(table, idx):
    v, d = table.shape
    b = idx.shape[0]
    info = plsc.get_sparse_core_info()
    nw = info.num_cores * info.num_subcores
    b_per_w = b // nw
    ch = 256
    n_ch = b_per_w // ch
    mesh = plsc.VectorSubcoreMesh(core_axis_name="c", subcore_axis_name="s")

    @functools.partial(
        pl.kernel, mesh=mesh,
        out_type=jax.ShapeDtypeStruct((b, d), jnp.float32),
        scratch_types=[
            pltpu.VMEM((ch,), jnp.int32),
            pltpu.VMEM((ch, d), jnp.float32),
            pltpu.SemaphoreType.DMA,
        ],
    )
    def gather_k(table_hbm, idx_hbm, out_hbm, idx_v, rows_v, sem):
        wid = lax.axis_index("s") * info.num_cores + lax.axis_index("c")
        base = wid * b_per_w
        for t in range(n_ch):
            off = base + t * ch
            pltpu.sync_copy(idx_hbm.at[pl.ds(off, ch)], idx_v)
            pltpu.async_copy(table_hbm.at[idx_v], rows_v, sem).wait()
            pltpu.sync_copy(rows_v, out_hbm.at[pl.ds(off, ch)])

    return gather_k(table, idx)


def kernel(plan, W_pre, b_pre, codebook):
    bsz, seq, k = plan.shape
    m, d = codebook.shape
    n = bsz * seq
    plan2 = plan.reshape(n, k)
    cbt = codebook.T
    b2 = b_pre.reshape(1, d)
    idx2, msum = _distance_argmin(plan2, W_pre, b2, cbt)
    idx_flat = idx2.reshape(n)
    zq = _sc_gather(codebook, idx_flat)
    commit = 1.25 * (jnp.sum(msum) / (n * d))
    return zq.reshape(bsz, seq, d), idx_flat.reshape(bsz, seq), commit
